# Initial kernel scaffold; baseline (speedup 1.0000x reference)
import jax
import jax.numpy as jnp
from jax.experimental import pallas as pl


def kernel(edge_index, edge_time, adj_time, centrality, W_time, b_time, W_node, b_node, att_a, weight):
    raise NotImplementedError("write your pallas kernel here")


# speedup vs baseline: 17.0946x; 17.0946x over previous
import functools

import numpy as np
import jax
import jax.numpy as jnp
from jax import lax
from jax.experimental import pallas as pl
from jax.experimental.pallas import tpu as pltpu
from jax.experimental.pallas import tpu_sc as plsc

N = 10000
E = 320000
D = 128
NC = 2
NS = 16
NW = NC * NS
EPT = E // NW
CH = 64
NCHUNK = EPT // CH
TAIL = EPT - NCHUNK * CH
WB = 40
NBLK = N // WB
BPS = -(-NBLK // NS)

_THRESHOLD = 50000.0
_I0 = np.int32(0)
_ALPHA = 0.2


def _node_feat_body(et_ref, at_ref, cen_ref, wt_ref, wn_ref, b_ref, a1_ref,
                    a2_ref, f_ref, u_ref, v_ref):
    t_ref = jnp.max(et_ref[...])
    dt = jnp.abs(t_ref - at_ref[...])
    pre = dt * wt_ref[...] + cen_ref[...] * wn_ref[...] + b_ref[...]
    r = jnp.maximum(pre, 0.0)
    nrm = jnp.sqrt(jnp.sum(r * r, axis=1, keepdims=True))
    f = r / jnp.maximum(nrm, 1e-12)
    f_ref[...] = f
    u_ref[...] = jnp.sum(f * a1_ref[...], axis=1, keepdims=True)
    v_ref[...] = jnp.sum(f * a2_ref[...], axis=1, keepdims=True)


def _node_feat(edge_time2d, adj_time, centrality, W_time, W_node, b, a1, a2):
    blk = 1000
    grid = (N // blk,)
    return pl.pallas_call(
        _node_feat_body,
        grid=grid,
        in_specs=[
            pl.BlockSpec(edge_time2d.shape, lambda i: (_I0, _I0)),
            pl.BlockSpec((blk, 1), lambda i: (i, _I0)),
            pl.BlockSpec((blk, 1), lambda i: (i, _I0)),
            pl.BlockSpec((1, D), lambda i: (_I0, _I0)),
            pl.BlockSpec((1, D), lambda i: (_I0, _I0)),
            pl.BlockSpec((1, D), lambda i: (_I0, _I0)),
            pl.BlockSpec((1, D), lambda i: (_I0, _I0)),
            pl.BlockSpec((1, D), lambda i: (_I0, _I0)),
        ],
        out_specs=[
            pl.BlockSpec((blk, D), lambda i: (i, _I0)),
            pl.BlockSpec((blk, 1), lambda i: (i, _I0)),
            pl.BlockSpec((blk, 1), lambda i: (i, _I0)),
        ],
        out_shape=[
            jax.ShapeDtypeStruct((N, D), jnp.float32),
            jax.ShapeDtypeStruct((N, 1), jnp.float32),
            jax.ShapeDtypeStruct((N, 1), jnp.float32),
        ],
    )(edge_time2d, adj_time, centrality, W_time, W_node, b, a1, a2)


def _edge_body(f_hbm, at_hbm, u_hbm, v_hbm, src_hbm, dst_hbm, et_hbm, out_hbm,
               at_v, u_v, v_v, rows0, rows1, srcb0, srcb1, dstb0, dstb1,
               etb0, etb1, dstc0, dstc1, dtail, accum,
               sg0, sg1, se0, se1, ss0, ss1):
    cid = lax.axis_index("c")
    sid = lax.axis_index("s")
    wid = cid * NS + sid
    rows = (rows0, rows1)
    srcb = (srcb0, srcb1)
    dstb = (dstb0, dstb1)
    etb = (etb0, etb1)
    dstc = (dstc0, dstc1)
    sg = (sg0, sg1)
    se = (se0, se1)
    ss = (ss0, ss1)

    pltpu.sync_copy(at_hbm, at_v)
    pltpu.sync_copy(u_hbm, u_v)
    pltpu.sync_copy(v_hbm, v_v)

    zero16 = jnp.zeros((16,), jnp.float32)

    def _zrow(b, carry):
        for j in range(D // 16):
            rows0[b, pl.ds(j * 16, 16)] = zero16
        return carry

    lax.fori_loop(jnp.int32(0), jnp.int32(CH), _zrow, jnp.int32(0))

    for k in range(BPS):
        m = sid * jnp.int32(BPS) + jnp.int32(k)

        @pl.when(m < NBLK)
        def _():
            pltpu.sync_copy(rows0.at[pl.ds(0, WB)],
                            accum.at[pl.ds(m * jnp.int32(WB), WB)])

    plsc.subcore_barrier()

    ebase = wid * jnp.int32(EPT)

    def start_edge(c, b):
        off = pl.ds(ebase + c * jnp.int32(CH), CH)
        pltpu.async_copy(src_hbm.at[off], srcb[b], se[b])
        pltpu.async_copy(dst_hbm.at[off], dstb[b], se[b])
        pltpu.async_copy(et_hbm.at[off], etb[b], se[b])

    def wait_edge(c, b):
        off = pl.ds(ebase + c * jnp.int32(CH), CH)
        pltpu.make_async_copy(src_hbm.at[off], srcb[b], se[b]).wait()
        pltpu.make_async_copy(dst_hbm.at[off], dstb[b], se[b]).wait()
        pltpu.make_async_copy(et_hbm.at[off], etb[b], se[b]).wait()

    H = CH // 2

    def start_gather(b):
        pltpu.async_copy(f_hbm.at[srcb[b].at[pl.ds(0, H)]],
                         rows[b].at[pl.ds(0, H)], sg[b])
        pltpu.async_copy(f_hbm.at[srcb[b].at[pl.ds(H, H)]],
                         rows[b].at[pl.ds(H, H)], sg[b])

    def wait_gather(b):
        pltpu.make_async_copy(f_hbm.at[srcb[b].at[pl.ds(0, H)]],
                              rows[b].at[pl.ds(0, H)], sg[b]).wait()
        pltpu.make_async_copy(f_hbm.at[srcb[b].at[pl.ds(H, H)]],
                              rows[b].at[pl.ds(H, H)], sg[b]).wait()

    def start_scatter(b):
        pltpu.async_copy(rows[b], accum.at[dstc[b]], ss[b], add=True)

    def wait_scatter(b):
        pltpu.make_async_copy(rows[b], accum.at[dstc[b]], ss[b]).wait()

    def compute(b):
        rb = rows[b]
        for i in range(CH // 16):
            sl = pl.ds(i * 16, 16)
            s16 = srcb[b][sl]
            d16 = dstb[b][sl]
            a = plsc.load_gather(at_v, [s16])
            uu = plsc.load_gather(u_v, [d16])
            vv = plsc.load_gather(v_v, [s16])
            dt = etb[b][sl] - a
            score = uu + vv
            score = jnp.where(score >= 0.0, score, _ALPHA * score)
            w = jnp.exp(jnp.abs(dt) * -0.5) * score
            w = jnp.where(dt <= _THRESHOLD, w, 0.0)
            dstc[b][sl] = d16
            for r in range(16):
                cb = w[r]
                brow = i * 16 + r
                for j in range(D // 16):
                    jsl = pl.ds(j * 16, 16)
                    rb[brow, jsl] = rb[brow, jsl] * cb

    def process(c, cur, first=False, last=False):
        nxt = 1 - cur
        if not last:
            start_edge(c + 1, nxt)
        wait_gather(cur)
        compute(cur)
        if not last:
            if not first:
                wait_scatter(nxt)
            wait_edge(c + 1, nxt)
            start_gather(nxt)
        start_scatter(cur)

    start_edge(jnp.int32(0), 0)
    wait_edge(jnp.int32(0), 0)
    start_gather(0)
    process(jnp.int32(0), 0, first=True)
    process(jnp.int32(1), 1)

    def _pair(t, carry):
        c = t * jnp.int32(2)
        process(c, 0)
        process(c + jnp.int32(1), 1)
        return carry

    if NCHUNK % 2 == 1:
        lax.fori_loop(jnp.int32(1), jnp.int32((NCHUNK - 1) // 2), _pair,
                      jnp.int32(0))
        process(jnp.int32(NCHUNK - 1), 0, last=True)
        wait_scatter(1)
        wait_scatter(0)
    else:
        lax.fori_loop(jnp.int32(1), jnp.int32((NCHUNK - 2) // 2), _pair,
                      jnp.int32(0))
        process(jnp.int32(NCHUNK - 2), 0)
        process(jnp.int32(NCHUNK - 1), 1, last=True)
        wait_scatter(0)
        wait_scatter(1)

    if TAIL:
        toff = pl.ds(ebase + jnp.int32(NCHUNK * CH), TAIL)
        tsl = pl.ds(0, TAIL)
        pltpu.sync_copy(src_hbm.at[toff], srcb0.at[tsl])
        pltpu.sync_copy(dst_hbm.at[toff], dstb0.at[tsl])
        pltpu.sync_copy(et_hbm.at[toff], etb0.at[tsl])
        pltpu.async_copy(f_hbm.at[srcb0.at[tsl]], rows0.at[tsl], sg0).wait()
        s16 = srcb0[tsl]
        d16 = dstb0[tsl]
        a = plsc.load_gather(at_v, [s16])
        uu = plsc.load_gather(u_v, [d16])
        vv = plsc.load_gather(v_v, [s16])
        dt = etb0[tsl] - a
        score = uu + vv
        score = jnp.where(score >= 0.0, score, _ALPHA * score)
        w = jnp.exp(jnp.abs(dt) * -0.5) * score
        w = jnp.where(dt <= _THRESHOLD, w, 0.0)
        dtail[tsl] = d16
        for r in range(TAIL):
            cb = w[r]
            for j in range(D // 16):
                jsl = pl.ds(j * 16, 16)
                rows0[r, jsl] = rows0[r, jsl] * cb
        pltpu.sync_copy(rows0.at[tsl], accum.at[dtail], add=True)

    plsc.subcore_barrier()

    for k in range(BPS):
        m = sid * jnp.int32(BPS) + jnp.int32(k)

        @pl.when(m < NBLK)
        def _():
            off = m * jnp.int32(WB)
            pltpu.sync_copy(accum.at[pl.ds(off, WB)], rows0.at[pl.ds(0, WB)])
            pltpu.sync_copy(rows0.at[pl.ds(0, WB)],
                            out_hbm.at[cid, pl.ds(off, WB)])


_EDGE_CALL = None


def _edge_call(*args):
    global _EDGE_CALL
    if _EDGE_CALL is None:
        _EDGE_CALL = pl.kernel(
            _edge_body,
            out_type=jax.ShapeDtypeStruct((NC, N, D), jnp.float32),
            mesh=plsc.VectorSubcoreMesh(
                core_axis_name="c", subcore_axis_name="s",
                num_cores=NC, num_subcores=NS),
            compiler_params=pltpu.CompilerParams(needs_layout_passes=False),
            scratch_types=[
                pltpu.VMEM((N,), jnp.float32),
                pltpu.VMEM((N,), jnp.float32),
                pltpu.VMEM((N,), jnp.float32),
                pltpu.VMEM((CH, D), jnp.float32),
                pltpu.VMEM((CH, D), jnp.float32),
                pltpu.VMEM((CH,), jnp.int32),
                pltpu.VMEM((CH,), jnp.int32),
                pltpu.VMEM((CH,), jnp.int32),
                pltpu.VMEM((CH,), jnp.int32),
                pltpu.VMEM((CH,), jnp.float32),
                pltpu.VMEM((CH,), jnp.float32),
                pltpu.VMEM((CH,), jnp.int32),
                pltpu.VMEM((CH,), jnp.int32),
                pltpu.VMEM((TAIL,), jnp.int32),
                pltpu.VMEM_SHARED((N, D), jnp.float32),
                pltpu.SemaphoreType.DMA,
                pltpu.SemaphoreType.DMA,
                pltpu.SemaphoreType.DMA,
                pltpu.SemaphoreType.DMA,
                pltpu.SemaphoreType.DMA,
                pltpu.SemaphoreType.DMA,
            ],
        )
    return _EDGE_CALL(*args)


def _combine_body(f_ref, p_ref, w_ref, o_ref):
    comb = f_ref[...] + p_ref[0] + p_ref[1]
    o_ref[...] = lax.dot_general(comb, w_ref[...], (((1,), (1,)), ((), ())),
                                 preferred_element_type=jnp.float32)


def _combine(f, partials, weight):
    blk = 2000
    grid = (N // blk,)
    return pl.pallas_call(
        _combine_body,
        grid=grid,
        in_specs=[
            pl.BlockSpec((blk, D), lambda i: (i, _I0)),
            pl.BlockSpec((NC, blk, D), lambda i: (_I0, i, _I0)),
            pl.BlockSpec(weight.shape, lambda i: (_I0, _I0)),
        ],
        out_specs=pl.BlockSpec((blk, weight.shape[0]), lambda i: (i, _I0)),
        out_shape=jax.ShapeDtypeStruct((N, weight.shape[0]), jnp.float32),
    )(f, partials, weight)


def kernel(edge_index, edge_time, adj_time, centrality, W_time, b_time,
           W_node, b_node, att_a, weight):
    src = edge_index[0].astype(jnp.int32)
    dst = edge_index[1].astype(jnp.int32)
    et = edge_time.astype(jnp.float32)
    b = (b_time + b_node).reshape(1, D).astype(jnp.float32)
    a1 = att_a[:D].reshape(1, D).astype(jnp.float32)
    a2 = att_a[D:].reshape(1, D).astype(jnp.float32)

    f, u, v = _node_feat(
        et.reshape(E // D, D),
        adj_time.reshape(N, 1).astype(jnp.float32),
        centrality.reshape(N, 1).astype(jnp.float32),
        W_time.astype(jnp.float32),
        W_node.astype(jnp.float32),
        b, a1, a2,
    )
    partials = _edge_call(
        f, adj_time.astype(jnp.float32), u.reshape(N), v.reshape(N),
        src, dst, et,
    )
    out = _combine(f, partials, weight.astype(jnp.float32))
    return out.astype(jnp.float64)
